# Initial kernel scaffold; baseline (speedup 1.0000x reference)
import jax
import jax.numpy as jnp
from jax.experimental import pallas as pl


def kernel(x, state, W_in, b_in, ln_g, ln_b, Wq, bq, Wk, bk, Wv, bv, Wo, bo, We, be, Wg, bg, Wr, br, Wsg, bsg, Wsp, bsp, Wout, bout):
    raise NotImplementedError("write your pallas kernel here")


# speedup vs baseline: 2.6279x; 2.6279x over previous
import jax
import jax.numpy as jnp
import numpy as np
from jax.experimental import pallas as pl
from jax.experimental.pallas import tpu as pltpu

B, T, D, NS, NE, TKS, TKE, NH = 8, 2048, 1024, 8, 16, 2, 2, 4
HD = D // NH
HS = NH * NS
INV = 1.0 / float(np.sqrt(HD))

_CT = (((1,), (1,)), ((), ()))
_F32 = jnp.float32


def _top2(vals, width):
    ii = jax.lax.broadcasted_iota(jnp.int32, vals.shape, len(vals.shape) - 1)
    m1 = jnp.max(vals, axis=-1, keepdims=True)
    i1 = jnp.min(jnp.where(vals == m1, ii, width), axis=-1, keepdims=True)
    masked = jnp.where(ii == i1, -jnp.inf, vals)
    m2 = jnp.max(masked, axis=-1, keepdims=True)
    i2 = jnp.min(jnp.where(masked == m2, ii, width), axis=-1, keepdims=True)
    return (jnp.concatenate([m1, m2], axis=-1),
            jnp.concatenate([i1, i2], axis=-1))


def _softmax2(v2):
    m = jnp.max(v2, axis=-1, keepdims=True)
    e = jnp.exp(v2 - m)
    return e / jnp.sum(e, axis=-1, keepdims=True)


def _prep_kernel(ss_ref, Win_ref, bin_ref, lng_ref, lnb_ref, Wq_ref, bq_ref,
                 Wk_ref, bk_ref, Wv_ref, bv_ref, Wr_ref, br_ref,
                 ridx_ref, qp_ref, cb_ref, zs_ref, vst_ref, bvf_ref):
    ss = ss_ref[...]
    Win = Win_ref[...]
    Wk = Wk_ref[...]
    bin2 = bin_ref[...]
    bk2 = bk_ref[...]

    rs = jnp.sum(ss * Wr_ref[...][None], axis=-1) + br_ref[0, 0]
    _, ridx = _top2(rs, NS)
    ridx_ref[...] = ridx

    mu = jnp.mean(ss, axis=-1, keepdims=True)
    c = ss - mu
    var = jnp.mean(c * c, axis=-1, keepdims=True)
    sn = c / jnp.sqrt(var + 1e-5) * lng_ref[...][None] + lnb_ref[...][None]
    snf = sn.reshape(B * NS, D)

    q = jax.lax.dot_general(snf, Wq_ref[...], _CT,
                            preferred_element_type=_F32) + bq_ref[...]
    kst = jax.lax.dot_general(snf, Wk, _CT,
                              preferred_element_type=_F32) + bk2
    vst = jax.lax.dot_general(snf, Wv_ref[...], _CT,
                              preferred_element_type=_F32) + bv_ref[...]
    vst_ref[...] = vst.reshape(B, NS, D)

    kb = jax.lax.dot_general(bin2, Wk, _CT,
                             preferred_element_type=_F32) + bk2
    bvf_ref[...] = jax.lax.dot_general(bin2, Wv_ref[...], _CT,
                                       preferred_element_type=_F32) + bv_ref[...]

    qps, cbs = [], []
    for h in range(NH):
        sl = slice(h * HD, (h + 1) * HD)
        qh = q[:, sl]
        tt = jnp.dot(qh, Wk[sl, :], preferred_element_type=_F32)
        qps.append(jnp.dot(tt, Win, preferred_element_type=_F32) * INV)
        cbs.append(jnp.sum(qh * kb[:, sl], axis=-1, keepdims=True) * INV)

    for b in range(B):
        rows = slice(b * NS, (b + 1) * NS)
        qp_ref[b] = jnp.concatenate([qps[h][rows] for h in range(NH)], axis=0)
        cb_ref[b] = jnp.concatenate([cbs[h][rows] for h in range(NH)], axis=0)
        zb = []
        for h in range(NH):
            sl = slice(h * HD, (h + 1) * HD)
            zb.append(jax.lax.dot_general(
                q[rows, sl], kst[rows, sl], _CT,
                preferred_element_type=_F32) * INV)
        zs_ref[b] = jnp.concatenate(zb, axis=0)


def _attn_kernel(x_ref, qp_ref, cb_ref, zs_ref, xbar_ref, wtok_ref, pst_ref):
    xb = x_ref[0]
    qp = qp_ref[0]
    zs = zs_ref[0]
    S = jax.lax.dot_general(qp, xb, _CT,
                            preferred_element_type=_F32) + cb_ref[0]
    m = jnp.maximum(jnp.max(zs, axis=-1, keepdims=True),
                    jnp.max(S, axis=-1, keepdims=True))
    P = jnp.exp(S - m)
    pst = jnp.exp(zs - m)
    ztok = jnp.sum(P, axis=-1, keepdims=True)
    Z = ztok + jnp.sum(pst, axis=-1, keepdims=True)
    A = jnp.dot(P, xb, preferred_element_type=_F32)
    xbar_ref[0] = A / Z
    wtok_ref[0] = ztok / Z
    pst_ref[0] = pst / Z


def _values_kernel(xbar_ref, wtok_ref, pst_ref, vst_ref, bvf_ref,
                   Win_ref, Wv_ref, Wo_ref, bo_ref, ao_ref):
    xf = xbar_ref[...].reshape(B * HS, D)
    U = jax.lax.dot_general(xf, Win_ref[...], _CT,
                            preferred_element_type=_F32)
    Yv = jax.lax.dot_general(U, Wv_ref[...], _CT,
                             preferred_element_type=_F32)
    wt = wtok_ref[...].reshape(B * HS, 1)
    stc = []
    for b in range(B):
        stc.append(jnp.dot(pst_ref[b], vst_ref[b],
                           preferred_element_type=_F32))
    st = jnp.concatenate(stc, axis=0)
    full = Yv + wt * bvf_ref[...] + st
    r = jax.lax.broadcasted_iota(jnp.int32, (B * HS, D), 0)
    d = jax.lax.broadcasted_iota(jnp.int32, (B * HS, D), 1)
    hmask = ((r % HS) // NS) == (d // HD)
    ao_pre = jnp.sum(jnp.where(hmask, full, 0.0).reshape(B, NH, NS, D), axis=1)
    ao = jax.lax.dot_general(ao_pre.reshape(B * NS, D), Wo_ref[...], _CT,
                             preferred_element_type=_F32) + bo_ref[...]
    ao_ref[...] = ao.reshape(B, NS, D)


_CT0 = (((0,), (0,)), ((), ()))


def _route_kernel(ao_ref, ss_ref, We_ref, be_ref, Wg_ref, bg_ref,
                  Wsg_ref, bsg_ref, Wsp_ref, bsp_ref,
                  eidx_ref, gw_ref, sidx_ref, sw_ref, ns_ref):
    ao = ao_ref[...]
    aof = ao.reshape(B * NS, D)
    ssf = ss_ref[...].reshape(B * NS, D)
    am = jnp.mean(ao, axis=1)
    gl = jax.lax.dot_general(am, Wg_ref[...], _CT,
                             preferred_element_type=_F32) + bg_ref[...]
    gval, eidx = _top2(gl, NE)
    gw = _softmax2(gval)
    eidx_ref[...] = eidx
    gw_ref[...] = gw
    gs = jnp.sum(gw, axis=-1, keepdims=True)

    r_i = jax.lax.broadcasted_iota(jnp.int32, (B * NS, B), 0)
    b_i = jax.lax.broadcasted_iota(jnp.int32, (B * NS, B), 1)
    rb = (r_i // NS == b_i).astype(_F32)
    smod = (jax.lax.broadcasted_iota(jnp.int32, (B * NS, 1), 0) % NS
            ).astype(_F32)

    gs_rows = jnp.dot(rb, gs, preferred_element_type=_F32)
    eof = (jax.lax.dot_general(aof, We_ref[...], _CT,
                               preferred_element_type=_F32)
           + be_ref[...]) * gs_rows
    sscore = jnp.sum(eof.reshape(B, NS, D) * Wsg_ref[...][None],
                     axis=-1) + bsg_ref[0, 0]
    sval, sidx = _top2(sscore, NS)
    sw = _softmax2(sval)
    sidx_ref[...] = sidx
    sw_ref[...] = sw

    sidxf = sidx.astype(_F32)
    ns = ssf
    for i in range(TKS):
        target = jnp.dot(rb, sidxf[:, i:i + 1],
                         preferred_element_type=_F32)
        rmask = (smod == target).astype(_F32)
        sel = jax.lax.dot_general(rb, eof * rmask, _CT0,
                                  preferred_element_type=_F32)
        upd = jnp.tanh(jax.lax.dot_general(sel, Wsp_ref[...], _CT,
                                           preferred_element_type=_F32)
                       + bsp_ref[...])
        ssel = jax.lax.dot_general(rb, ssf * rmask, _CT0,
                                   preferred_element_type=_F32)
        newv = 0.7 * ssel + 0.3 * sw[:, i:i + 1] * upd
        ns = ns * (1.0 - rmask) + jnp.dot(rb, newv,
                                          preferred_element_type=_F32) * rmask
    ns_ref[...] = ns.reshape(B, NS, D)


def _out_kernel(nsf_ref, Wout_ref, bout_ref, out_ref):
    c = pl.program_id(0)
    part = jax.lax.dot_general(nsf_ref[...], Wout_ref[...], _CT,
                               preferred_element_type=_F32)

    @pl.when(c == 0)
    def _():
        out_ref[...] = part + bout_ref[...]

    @pl.when(c != 0)
    def _():
        out_ref[...] += part


def kernel(x, state, W_in, b_in, ln_g, ln_b, Wq, bq, Wk, bk, Wv, bv, Wo, bo,
           We, be, Wg, bg, Wr, br, Wsg, bsg, Wsp, bsp, Wout, bout):
    f32 = jnp.float32
    ss = state.reshape(B, NS, D)
    b2 = lambda v: v.reshape(1, -1)

    ridx, qp, cb, zs, vst, bvf = pl.pallas_call(
        _prep_kernel,
        out_shape=(
            jax.ShapeDtypeStruct((B, TKS), jnp.int32),
            jax.ShapeDtypeStruct((B, HS, D), f32),
            jax.ShapeDtypeStruct((B, HS, 1), f32),
            jax.ShapeDtypeStruct((B, HS, NS), f32),
            jax.ShapeDtypeStruct((B, NS, D), f32),
            jax.ShapeDtypeStruct((1, D), f32),
        ),
    )(ss, W_in, b2(b_in), b2(ln_g), b2(ln_b), Wq, b2(bq), Wk, b2(bk),
      Wv, b2(bv), Wr, b2(br))

    xbar, wtok, pst = pl.pallas_call(
        _attn_kernel,
        grid=(B,),
        in_specs=[
            pl.BlockSpec((1, T, D), lambda b: (b, 0, 0)),
            pl.BlockSpec((1, HS, D), lambda b: (b, 0, 0)),
            pl.BlockSpec((1, HS, 1), lambda b: (b, 0, 0)),
            pl.BlockSpec((1, HS, NS), lambda b: (b, 0, 0)),
        ],
        out_specs=(
            pl.BlockSpec((1, HS, D), lambda b: (b, 0, 0)),
            pl.BlockSpec((1, HS, 1), lambda b: (b, 0, 0)),
            pl.BlockSpec((1, HS, NS), lambda b: (b, 0, 0)),
        ),
        out_shape=(
            jax.ShapeDtypeStruct((B, HS, D), f32),
            jax.ShapeDtypeStruct((B, HS, 1), f32),
            jax.ShapeDtypeStruct((B, HS, NS), f32),
        ),
    )(x, qp, cb, zs)

    ao = pl.pallas_call(
        _values_kernel,
        out_shape=jax.ShapeDtypeStruct((B, NS, D), f32),
    )(xbar, wtok, pst, vst, bvf, W_in, Wv, Wo, b2(bo))

    eidx, gw, sidx, sw, ns = pl.pallas_call(
        _route_kernel,
        out_shape=(
            jax.ShapeDtypeStruct((B, TKE), jnp.int32),
            jax.ShapeDtypeStruct((B, TKE), f32),
            jax.ShapeDtypeStruct((B, TKS), jnp.int32),
            jax.ShapeDtypeStruct((B, TKS), f32),
            jax.ShapeDtypeStruct((B, NS, D), f32),
        ),
    )(ao, ss, We, b2(be), Wg, b2(bg), Wsg, b2(bsg), Wsp, b2(bsp))

    nsf = ns.reshape(B, NS * D)
    KC = NS
    out = pl.pallas_call(
        _out_kernel,
        grid=(KC,),
        in_specs=[
            pl.BlockSpec((B, D), lambda c: (0, c)),
            pl.BlockSpec((Wout.shape[0], D), lambda c: (0, c)),
            pl.BlockSpec((1, Wout.shape[0]), lambda c: (0, 0)),
        ],
        out_specs=pl.BlockSpec((B, Wout.shape[0]), lambda c: (0, 0)),
        out_shape=jax.ShapeDtypeStruct((B, Wout.shape[0]), f32),
    )(nsf, Wout, b2(bout))

    return out, ridx, eidx, sidx, sw, gw, nsf
